# baseline (speedup 1.0000x reference)
import functools

import jax
import jax.numpy as jnp
from jax import lax
from jax.experimental import pallas as pl
from jax.experimental.pallas import tpu as pltpu
from jax.experimental.pallas import tpu_sc as plsc

N = 10000
E = 160000
F = 128
S = 27
SB = 32

NC = 2
NS = 16
L = 16
FH = F // NC

RL = 1792
CH = 16
EPT = E // NS
CPT = EPT // CH

NP = 10240
RPT = NP // NS


def _mm_body(x_ref, wt_ref, y_ref):
    y_ref[...] = jnp.dot(x_ref[...], wt_ref[0],
                         preferred_element_type=jnp.float32,
                         precision=lax.Precision.HIGHEST).astype(jnp.bfloat16)


def _compute_y(x, wt):
    blk = 2000
    return pl.pallas_call(
        _mm_body,
        grid=(NC, N // blk),
        in_specs=[
            pl.BlockSpec((blk, F), lambda c, i: (i, 0)),
            pl.BlockSpec((1, F, RL), lambda c, i: (c, 0, 0)),
        ],
        out_specs=pl.BlockSpec((blk, RL), lambda c, i: (c * (N // blk) + i, 0)),
        out_shape=jax.ShapeDtypeStruct((NC * N, RL), jnp.bfloat16),
    )(x, wt)


def _compute_basis(edge_attr):
    blk = 16000
    return pl.pallas_call(
        _basis_math,
        grid=(E // blk,),
        in_specs=[pl.BlockSpec((blk, 3), lambda i: (i, 0))],
        out_specs=pl.BlockSpec((blk, SB), lambda i: (i, 0)),
        out_shape=jax.ShapeDtypeStruct((E, SB), jnp.float32),
    )(edge_attr)


def _basis_math(attr_ref, out_ref):
    a = attr_ref[...]
    g = 0.5 * jnp.concatenate(
        [(1.0 - a) ** 2, -2.0 * a * a + 2.0 * a + 1.0, a * a], axis=1
    )
    col = lax.broadcasted_iota(jnp.int32, (9, SB), 1)
    row = lax.broadcasted_iota(jnp.int32, (9, SB), 0)
    valid = col < S
    q0 = jnp.where(valid & (row == (col // 9) * 3), 1.0, 0.0)
    q1 = jnp.where(valid & (row == ((col // 3) % 3) * 3 + 1), 1.0, 0.0)
    q2 = jnp.where(valid & (row == (col % 3) * 3 + 2), 1.0, 0.0)
    p0 = jnp.dot(g, q0, preferred_element_type=jnp.float32)
    p1 = jnp.dot(g, q1, preferred_element_type=jnp.float32)
    p2 = jnp.dot(g, q2, preferred_element_type=jnp.float32)
    out_ref[...] = p0 * p1 * p2


SPC = 5
NSUP = CPT // SPC
EB = SPC * CH


def _sc_body(y_hbm, srcs_hbm, dsts_hbm, basis_hbm, zf_hbm, z16_hbm,
             outp_hbm, degp_hbm,
             srcb2, dstb2, idxb2, bvb2, rows2, zb2, onesb, acc, dacc,
             gsem, lsem, ssem, dsem):
    cid = lax.axis_index("c")
    sid = lax.axis_index("s")

    r0 = sid * RPT
    pltpu.sync_copy(zf_hbm.at[pl.ds(r0, RPT)], acc.at[pl.ds(r0, RPT)])

    @pl.when(cid == 0)
    def _():
        pltpu.sync_copy(z16_hbm.at[pl.ds(r0, RPT)], dacc.at[pl.ds(r0, RPT)])

    def ones_body(i, carry):
        onesb[i] = jnp.ones((L,), jnp.float32)
        return carry

    lax.fori_loop(0, EB, ones_body, 0)

    ebase0 = sid * EPT
    pltpu.sync_copy(srcs_hbm.at[pl.ds(ebase0, EB)], srcb2.at[0])
    pltpu.sync_copy(dsts_hbm.at[pl.ds(ebase0, EB)], dstb2.at[0])
    pltpu.sync_copy(basis_hbm.at[pl.ds(ebase0, EB)], bvb2.at[0])
    for k in range(SPC):
        idxb2[0, k] = srcb2[0, pl.ds(k * CH, CH)] + cid * N

    plsc.subcore_barrier()

    def sup_body(t, carry):
        tp = lax.rem(t, 2)
        tq = 1 - tp

        @pl.when(t + 1 < NSUP)
        def _():
            ebase = sid * EPT + (t + 1) * EB
            pltpu.async_copy(srcs_hbm.at[pl.ds(ebase, EB)], srcb2.at[tq],
                             lsem.at[tq])
            pltpu.async_copy(dsts_hbm.at[pl.ds(ebase, EB)], dstb2.at[tq],
                             lsem.at[tq])
            pltpu.async_copy(basis_hbm.at[pl.ds(ebase, EB)], bvb2.at[tq],
                             lsem.at[tq])

        @pl.when(t >= 2)
        def _():
            pltpu.make_async_copy(zb2.at[tp], acc.at[dstb2.at[tp]],
                                  ssem.at[tp]).wait()

            @pl.when(cid == 0)
            def _():
                pltpu.make_async_copy(onesb, dacc.at[dstb2.at[tp]],
                                      dsem.at[tp]).wait()

        descs = [None] * SPC
        descs[0] = pltpu.async_copy(y_hbm.at[idxb2.at[tp, 0]], rows2.at[0],
                                    gsem.at[0])
        for g in range(SPC):
            p = g % 2
            if g + 1 < SPC:
                descs[g + 1] = pltpu.async_copy(
                    y_hbm.at[idxb2.at[tp, g + 1]], rows2.at[1 - p],
                    gsem.at[1 - p])
            descs[g].wait()
            goff = g * CH
            rows = rows2.at[p]

            def emit_edge(c, goff=goff, rows=rows):
                e = goff + c
                brow = [bvb2[tp, e, pl.ds(0, L)], bvb2[tp, e, pl.ds(L, L)]]
                acc0 = [None] * (FH // L)
                acc1 = [None] * (FH // L)
                for s in range(S):
                    b = brow[s // L][s % L]
                    for j in range(FH // (2 * L)):
                        ab = rows[c, pl.ds(s * FH + j * 2 * L, 2 * L)]
                        va, vb = plsc.unpack(
                            ab, format=plsc.PackFormat.INTERLEAVED,
                            preferred_element_type=jnp.float32)
                        dst = acc0 if s % 2 == 0 else acc1
                        if dst[2 * j] is None:
                            dst[2 * j] = b * va
                            dst[2 * j + 1] = b * vb
                        else:
                            dst[2 * j] = dst[2 * j] + b * va
                            dst[2 * j + 1] = dst[2 * j + 1] + b * vb
                for fb in range(FH // L):
                    zb2[tp, e, pl.ds(fb * L, L)] = acc0[fb] + acc1[fb]

            def edge_body(c2, carry2):
                emit_edge(2 * c2)
                emit_edge(2 * c2 + 1)
                return carry2

            lax.fori_loop(0, CH // 2, edge_body, 0)

        pltpu.async_copy(zb2.at[tp], acc.at[dstb2.at[tp]], ssem.at[tp],
                         add=True)

        @pl.when(cid == 0)
        def _():
            pltpu.async_copy(onesb, dacc.at[dstb2.at[tp]], dsem.at[tp],
                             add=True)

        @pl.when(t + 1 < NSUP)
        def _():
            ebase = sid * EPT + (t + 1) * EB
            pltpu.make_async_copy(srcs_hbm.at[pl.ds(ebase, EB)],
                                  srcb2.at[tq], lsem.at[tq]).wait()
            pltpu.make_async_copy(dsts_hbm.at[pl.ds(ebase, EB)],
                                  dstb2.at[tq], lsem.at[tq]).wait()
            pltpu.make_async_copy(basis_hbm.at[pl.ds(ebase, EB)],
                                  bvb2.at[tq], lsem.at[tq]).wait()
            for k in range(SPC):
                idxb2[tq, k] = srcb2[tq, pl.ds(k * CH, CH)] + cid * N

        return carry

    lax.fori_loop(0, NSUP, sup_body, 0)

    for tp in range(2):
        pltpu.make_async_copy(zb2.at[tp], acc.at[dstb2.at[tp]],
                              ssem.at[tp]).wait()

    @pl.when(cid == 0)
    def _():
        for tp in range(2):
            pltpu.make_async_copy(onesb, dacc.at[dstb2.at[tp]],
                                  dsem.at[tp]).wait()

    plsc.subcore_barrier()
    pltpu.sync_copy(acc.at[pl.ds(r0, RPT)], outp_hbm.at[cid, pl.ds(r0, RPT)])

    @pl.when(cid == 0)
    def _():
        pltpu.sync_copy(dacc.at[pl.ds(r0, RPT)], degp_hbm.at[pl.ds(r0, RPT)])


@functools.cache
def _sc_scatter_kernel():
  return functools.partial(
    pl.kernel,
    out_type=[
        jax.ShapeDtypeStruct((NC, NP, FH), jnp.float32),
        jax.ShapeDtypeStruct((NP, L), jnp.float32),
    ],
    mesh=plsc.VectorSubcoreMesh(core_axis_name="c", subcore_axis_name="s",
                                num_cores=NC, num_subcores=NS),
    scratch_types=[
        pltpu.VMEM((2, EB), jnp.int32),
        pltpu.VMEM((2, EB), jnp.int32),
        pltpu.VMEM((2, SPC, CH), jnp.int32),
        pltpu.VMEM((2, EB, SB), jnp.float32),
        pltpu.VMEM((2, CH, RL), jnp.bfloat16),
        pltpu.VMEM((2, EB, FH), jnp.float32),
        pltpu.VMEM((EB, L), jnp.float32),
        pltpu.VMEM_SHARED((NP, FH), jnp.float32),
        pltpu.VMEM_SHARED((NP, L), jnp.float32),
        pltpu.SemaphoreType.DMA((2,)),
        pltpu.SemaphoreType.DMA((2,)),
        pltpu.SemaphoreType.DMA((2,)),
        pltpu.SemaphoreType.DMA((2,)),
    ],
    compiler_params=pltpu.CompilerParams(use_tc_tiling_on_sc=False,
                                         needs_layout_passes=False),
  )(_sc_body)


def _combine_body(p_ref, d_ref, out_ref):
    inv = 1.0 / jnp.maximum(d_ref[:, 0:1], 1.0)
    out_ref[...] = jnp.concatenate([p_ref[0], p_ref[1]], axis=1) * inv


def _combine(partials, degs):
    blk = 1024
    return pl.pallas_call(
        _combine_body,
        grid=(NP // blk,),
        in_specs=[
            pl.BlockSpec((NC, blk, FH), lambda i: (0, i, 0)),
            pl.BlockSpec((blk, L), lambda i: (i, 0)),
        ],
        out_specs=pl.BlockSpec((blk, F), lambda i: (i, 0)),
        out_shape=jax.ShapeDtypeStruct((NP, F), jnp.float32),
    )(partials, degs)


def kernel(x, edge_index, edge_attr, W):
    src = edge_index[0]
    dst = edge_index[1]
    wt = (jnp.transpose(W, (1, 0, 2))
          .reshape(F, S, NC, FH)
          .transpose(0, 2, 1, 3)
          .reshape(F, NC, S, 2, 2, 16)
          .transpose(0, 1, 2, 3, 5, 4)
          .reshape(F, NC, S * FH))
    wt = jnp.pad(wt, ((0, 0), (0, 0), (0, RL - S * FH))).transpose(1, 0, 2)

    y2 = _compute_y(x, wt)
    basis = _compute_basis(edge_attr)

    zf = jnp.zeros((NP, FH), jnp.float32)
    z16 = jnp.zeros((NP, L), jnp.float32)
    partials, degs = _sc_scatter_kernel()(y2, src, dst, basis, zf, z16)
    return _combine(partials, degs)[:N]


# speedup vs baseline: 1.0613x; 1.0613x over previous
import functools

import jax
import jax.numpy as jnp
from jax import lax
from jax.experimental import pallas as pl
from jax.experimental.pallas import tpu as pltpu
from jax.experimental.pallas import tpu_sc as plsc

N = 10000
E = 160000
F = 128
S = 27
SB = 32

NC = 2
NS = 16
L = 16
FH = F // NC

RL = 1792
CH = 16
EPT = E // NS
CPT = EPT // CH

NP = 10240
RPT = NP // NS


def _mm_body(x_ref, wt_ref, y_ref):
    y_ref[...] = jnp.dot(x_ref[...], wt_ref[0],
                         preferred_element_type=jnp.float32).astype(jnp.bfloat16)


def _compute_y(x, wt):
    blk = 2000
    return pl.pallas_call(
        _mm_body,
        grid=(NC, N // blk),
        in_specs=[
            pl.BlockSpec((blk, F), lambda c, i: (i, 0)),
            pl.BlockSpec((1, F, RL), lambda c, i: (c, 0, 0)),
        ],
        out_specs=pl.BlockSpec((blk, RL), lambda c, i: (c * (N // blk) + i, 0)),
        out_shape=jax.ShapeDtypeStruct((NC * N, RL), jnp.bfloat16),
    )(x, wt)


def _compute_basis(edge_attr):
    blk = 16000
    return pl.pallas_call(
        _basis_math,
        grid=(E // blk,),
        in_specs=[pl.BlockSpec((blk, 3), lambda i: (i, 0))],
        out_specs=pl.BlockSpec((blk, SB), lambda i: (i, 0)),
        out_shape=jax.ShapeDtypeStruct((E, SB), jnp.float32),
    )(edge_attr)


def _basis_math(attr_ref, out_ref):
    a = attr_ref[...]
    g = 0.5 * jnp.concatenate(
        [(1.0 - a) ** 2, -2.0 * a * a + 2.0 * a + 1.0, a * a], axis=1
    )
    col = lax.broadcasted_iota(jnp.int32, (9, SB), 1)
    row = lax.broadcasted_iota(jnp.int32, (9, SB), 0)
    valid = col < S
    q0 = jnp.where(valid & (row == (col // 9) * 3), 1.0, 0.0)
    q1 = jnp.where(valid & (row == ((col // 3) % 3) * 3 + 1), 1.0, 0.0)
    q2 = jnp.where(valid & (row == (col % 3) * 3 + 2), 1.0, 0.0)
    p0 = jnp.dot(g, q0, preferred_element_type=jnp.float32)
    p1 = jnp.dot(g, q1, preferred_element_type=jnp.float32)
    p2 = jnp.dot(g, q2, preferred_element_type=jnp.float32)
    out_ref[...] = p0 * p1 * p2


SPC = 5
NSUP = CPT // SPC
EB = SPC * CH


def _sc_body(y_hbm, srcs_hbm, dsts_hbm, basis_hbm, zf_hbm, z16_hbm,
             outp_hbm, degp_hbm,
             srcb2, dstb2, idxb2, bvb2, rows2, zb2, onesb, acc, dacc,
             gsem, lsem, ssem, dsem):
    cid = lax.axis_index("c")
    sid = lax.axis_index("s")

    r0 = sid * RPT
    pltpu.sync_copy(zf_hbm.at[pl.ds(r0, RPT)], acc.at[pl.ds(r0, RPT)])

    @pl.when(cid == 0)
    def _():
        pltpu.sync_copy(z16_hbm.at[pl.ds(r0, RPT)], dacc.at[pl.ds(r0, RPT)])

    def ones_body(i, carry):
        onesb[i] = jnp.ones((L,), jnp.float32)
        return carry

    lax.fori_loop(0, EB, ones_body, 0)

    ebase0 = sid * EPT
    pltpu.sync_copy(srcs_hbm.at[pl.ds(ebase0, EB)], srcb2.at[0])
    pltpu.sync_copy(dsts_hbm.at[pl.ds(ebase0, EB)], dstb2.at[0])
    pltpu.sync_copy(basis_hbm.at[pl.ds(ebase0, EB)], bvb2.at[0])
    for k in range(SPC):
        idxb2[0, k] = srcb2[0, pl.ds(k * CH, CH)] + cid * N

    plsc.subcore_barrier()

    def sup_body(t, carry):
        tp = lax.rem(t, 2)
        tq = 1 - tp

        @pl.when(t + 1 < NSUP)
        def _():
            ebase = sid * EPT + (t + 1) * EB
            pltpu.async_copy(srcs_hbm.at[pl.ds(ebase, EB)], srcb2.at[tq],
                             lsem.at[tq])
            pltpu.async_copy(dsts_hbm.at[pl.ds(ebase, EB)], dstb2.at[tq],
                             lsem.at[tq])
            pltpu.async_copy(basis_hbm.at[pl.ds(ebase, EB)], bvb2.at[tq],
                             lsem.at[tq])

        @pl.when(t >= 2)
        def _():
            pltpu.make_async_copy(zb2.at[tp], acc.at[dstb2.at[tp]],
                                  ssem.at[tp]).wait()

            @pl.when(cid == 0)
            def _():
                pltpu.make_async_copy(onesb, dacc.at[dstb2.at[tp]],
                                      dsem.at[tp]).wait()

        descs = [None] * SPC
        descs[0] = pltpu.async_copy(y_hbm.at[idxb2.at[tp, 0]], rows2.at[0],
                                    gsem.at[0])
        for g in range(SPC):
            p = g % 2
            if g + 1 < SPC:
                descs[g + 1] = pltpu.async_copy(
                    y_hbm.at[idxb2.at[tp, g + 1]], rows2.at[1 - p],
                    gsem.at[1 - p])
            descs[g].wait()
            goff = g * CH
            rows = rows2.at[p]

            def emit_edge(c, goff=goff, rows=rows):
                e = goff + c
                brow = [bvb2[tp, e, pl.ds(0, L)], bvb2[tp, e, pl.ds(L, L)]]
                acc0 = [None] * (FH // L)
                acc1 = [None] * (FH // L)
                for s in range(S):
                    b = brow[s // L][s % L]
                    for j in range(FH // (2 * L)):
                        ab = rows[c, pl.ds(s * FH + j * 2 * L, 2 * L)]
                        va, vb = plsc.unpack(
                            ab, format=plsc.PackFormat.INTERLEAVED,
                            preferred_element_type=jnp.float32)
                        dst = acc0 if s % 2 == 0 else acc1
                        if dst[2 * j] is None:
                            dst[2 * j] = b * va
                            dst[2 * j + 1] = b * vb
                        else:
                            dst[2 * j] = dst[2 * j] + b * va
                            dst[2 * j + 1] = dst[2 * j + 1] + b * vb
                for fb in range(FH // L):
                    zb2[tp, e, pl.ds(fb * L, L)] = acc0[fb] + acc1[fb]

            def edge_body(c2, carry2):
                emit_edge(2 * c2)
                emit_edge(2 * c2 + 1)
                return carry2

            lax.fori_loop(0, CH // 2, edge_body, 0)

        pltpu.async_copy(zb2.at[tp], acc.at[dstb2.at[tp]], ssem.at[tp],
                         add=True)

        @pl.when(cid == 0)
        def _():
            pltpu.async_copy(onesb, dacc.at[dstb2.at[tp]], dsem.at[tp],
                             add=True)

        @pl.when(t + 1 < NSUP)
        def _():
            ebase = sid * EPT + (t + 1) * EB
            pltpu.make_async_copy(srcs_hbm.at[pl.ds(ebase, EB)],
                                  srcb2.at[tq], lsem.at[tq]).wait()
            pltpu.make_async_copy(dsts_hbm.at[pl.ds(ebase, EB)],
                                  dstb2.at[tq], lsem.at[tq]).wait()
            pltpu.make_async_copy(basis_hbm.at[pl.ds(ebase, EB)],
                                  bvb2.at[tq], lsem.at[tq]).wait()
            for k in range(SPC):
                idxb2[tq, k] = srcb2[tq, pl.ds(k * CH, CH)] + cid * N

        return carry

    lax.fori_loop(0, NSUP, sup_body, 0)

    for tp in range(2):
        pltpu.make_async_copy(zb2.at[tp], acc.at[dstb2.at[tp]],
                              ssem.at[tp]).wait()

    @pl.when(cid == 0)
    def _():
        for tp in range(2):
            pltpu.make_async_copy(onesb, dacc.at[dstb2.at[tp]],
                                  dsem.at[tp]).wait()

    plsc.subcore_barrier()
    pltpu.sync_copy(acc.at[pl.ds(r0, RPT)], outp_hbm.at[cid, pl.ds(r0, RPT)])

    @pl.when(cid == 0)
    def _():
        pltpu.sync_copy(dacc.at[pl.ds(r0, RPT)], degp_hbm.at[pl.ds(r0, RPT)])


@functools.cache
def _sc_scatter_kernel():
  return functools.partial(
    pl.kernel,
    out_type=[
        jax.ShapeDtypeStruct((NC, NP, FH), jnp.float32),
        jax.ShapeDtypeStruct((NP, L), jnp.float32),
    ],
    mesh=plsc.VectorSubcoreMesh(core_axis_name="c", subcore_axis_name="s",
                                num_cores=NC, num_subcores=NS),
    scratch_types=[
        pltpu.VMEM((2, EB), jnp.int32),
        pltpu.VMEM((2, EB), jnp.int32),
        pltpu.VMEM((2, SPC, CH), jnp.int32),
        pltpu.VMEM((2, EB, SB), jnp.float32),
        pltpu.VMEM((2, CH, RL), jnp.bfloat16),
        pltpu.VMEM((2, EB, FH), jnp.float32),
        pltpu.VMEM((EB, L), jnp.float32),
        pltpu.VMEM_SHARED((NP, FH), jnp.float32),
        pltpu.VMEM_SHARED((NP, L), jnp.float32),
        pltpu.SemaphoreType.DMA((2,)),
        pltpu.SemaphoreType.DMA((2,)),
        pltpu.SemaphoreType.DMA((2,)),
        pltpu.SemaphoreType.DMA((2,)),
    ],
    compiler_params=pltpu.CompilerParams(use_tc_tiling_on_sc=False,
                                         needs_layout_passes=False),
  )(_sc_body)


def _combine_body(p_ref, d_ref, out_ref):
    inv = 1.0 / jnp.maximum(d_ref[:, 0:1], 1.0)
    out_ref[...] = jnp.concatenate([p_ref[0], p_ref[1]], axis=1) * inv


def _combine(partials, degs):
    blk = 1024
    return pl.pallas_call(
        _combine_body,
        grid=(NP // blk,),
        in_specs=[
            pl.BlockSpec((NC, blk, FH), lambda i: (0, i, 0)),
            pl.BlockSpec((blk, L), lambda i: (i, 0)),
        ],
        out_specs=pl.BlockSpec((blk, F), lambda i: (i, 0)),
        out_shape=jax.ShapeDtypeStruct((NP, F), jnp.float32),
    )(partials, degs)


def kernel(x, edge_index, edge_attr, W):
    src = edge_index[0]
    dst = edge_index[1]
    wt = (jnp.transpose(W, (1, 0, 2))
          .reshape(F, S, NC, FH)
          .transpose(0, 2, 1, 3)
          .reshape(F, NC, S, 2, 2, 16)
          .transpose(0, 1, 2, 3, 5, 4)
          .reshape(F, NC, S * FH))
    wt = jnp.pad(wt, ((0, 0), (0, 0), (0, RL - S * FH))).transpose(1, 0, 2)

    y2 = _compute_y(x, wt)
    basis = _compute_basis(edge_attr)

    zf = jnp.zeros((NP, FH), jnp.float32)
    z16 = jnp.zeros((NP, L), jnp.float32)
    partials, degs = _sc_scatter_kernel()(y2, src, dst, basis, zf, z16)
    return _combine(partials, degs)[:N]
